# baseline (speedup 1.0000x reference)
import functools

import jax
import jax.numpy as jnp
from jax import lax
from jax.experimental import pallas as pl
from jax.experimental.pallas import tpu as pltpu
from jax.experimental.pallas import tpu_sc as plsc

B, R, C = 4, 4096, 2048
ROWS = B * R
NW = 32
BAND = 8


def _take16(v, idx):
    return lax.gather(
        v,
        idx[:, None],
        dimension_numbers=lax.GatherDimensionNumbers(
            offset_dims=(), collapsed_slice_dims=(0,), start_index_map=(0,)),
        slice_sizes=(1,),
        mode=lax.GatherScatterMode.PROMISE_IN_BOUNDS,
    )


def _build_sc(row_lo, nrows, interpret=False):
    rows_w = nrows // NW
    nchunks = rows_w // BAND
    steps = BAND * C // 32

    @functools.partial(
        pl.kernel,
        out_type=jax.ShapeDtypeStruct((nrows, C), jnp.float32),
        mesh=plsc.VectorSubcoreMesh(core_axis_name="c", subcore_axis_name="s"),
        scratch_types=[
            pltpu.VMEM((BAND, C), jnp.float32),
            pltpu.VMEM((BAND, C), jnp.float32),
            pltpu.VMEM((BAND, C), jnp.float32),
            pltpu.VMEM((BAND, C), jnp.float32),
            pltpu.SemaphoreType.DMA,
            pltpu.SemaphoreType.DMA,
            pltpu.SemaphoreType.DMA,
            pltpu.SemaphoreType.DMA,
        ],
        compiler_params=pltpu.CompilerParams(
            needs_layout_passes=False, use_tc_tiling_on_sc=True),
        interpret=interpret,
    )
    def sc_mask48(x_hbm, out_hbm, xin0, xin1, xout0, xout1,
                  si0, si1, so0, so1):
        cid = lax.axis_index("c")
        sid = lax.axis_index("s")
        wid = sid * 2 + cid
        row0 = wid * rows_w
        xin = (xin0, xin1)
        xout = (xout0, xout1)
        sem_in = (si0, si1)
        sem_out = (so0, so1)

        lane = lax.iota(jnp.int32, 16)
        q = lane & 3
        rots = [(lane - q) + ((q + k) & 3) for k in (1, 2, 3)]
        idx_e = lane * 2
        idx_o = idx_e + 1
        zero = jnp.zeros((16,), jnp.float32)
        zeroi = jnp.zeros((16,), jnp.int32)

        def make_step(src, dst):
            def step(j, carry):
                r = j // (C // 32)
                cb = (j % (C // 32)) * 32
                rvec = zeroi + r
                ie = cb + idx_e
                io = cb + idx_o
                a = plsc.load_gather(src, [rvec, ie])
                b = plsc.load_gather(src, [rvec, io])
                sq = a * a + b * b
                c1 = _take16(sq, rots[0]) < sq
                c2 = _take16(sq, rots[1]) < sq
                c3 = _take16(sq, rots[2]) < sq
                keep = (c1 & c2) | (c1 & c3) | (c2 & c3)
                plsc.store_scatter(dst, [rvec, ie], jnp.where(keep, a, zero))
                plsc.store_scatter(dst, [rvec, io], jnp.where(keep, b, zero))
                return carry
            return step

        def in_slice(ci):
            return x_hbm.at[pl.ds(row_lo + row0 + ci * BAND, BAND)]

        def out_slice(ci):
            return out_hbm.at[pl.ds(row0 + ci * BAND, BAND)]

        for b in (0, 1):
            pltpu.async_copy(in_slice(b), xin[b], sem_in[b])

        def chunk_pair(cp, carry):
            for b in (0, 1):
                ci = cp * 2 + b
                pltpu.make_async_copy(in_slice(ci), xin[b], sem_in[b]).wait()
                @pl.when(ci >= 2)
                def _():
                    pltpu.make_async_copy(xout[b], out_slice(ci),
                                          sem_out[b]).wait()
                step_fn = make_step(xin[b], xout[b])
                plsc.parallel_loop(0, steps, 1, unroll=4)(
                    lambda j, fn=step_fn: fn(j, None))
                pltpu.async_copy(xout[b], out_slice(ci), sem_out[b])
                @pl.when(ci + 2 < nchunks)
                def _():
                    pltpu.async_copy(in_slice(ci + 2), xin[b], sem_in[b])
            return carry

        lax.fori_loop(0, nchunks // 2, chunk_pair, 0)

        for b in (0, 1):
            pltpu.make_async_copy(xout[b], out_slice(nchunks - 2 + b),
                                  sem_out[b]).wait()

    return sc_mask48


SPLIT = 12288
_sc_a = _build_sc(0, SPLIT)
_sc_b = _build_sc(SPLIT, ROWS - SPLIT)


def kernel(x):
    x2 = x.reshape(ROWS, C)
    a = _sc_a(x2)
    b = _sc_b(x2)
    return jnp.concatenate([a, b], axis=0).reshape(x.shape)
